# Initial kernel scaffold; baseline (speedup 1.0000x reference)
import jax
import jax.numpy as jnp
from jax.experimental import pallas as pl


def kernel(hidden_states, router_weight, gate_proj, up_proj, down_proj):
    raise NotImplementedError("write your pallas kernel here")


# speedup vs baseline: 1.8615x; 1.8615x over previous
import functools

import jax
import jax.numpy as jnp
from jax.experimental import pallas as pl
from jax.experimental.pallas import tpu as pltpu

E = 8
TOP_K = 2
H = 1024
F = 768
N = 2048


def _router_body(x_ref, rw_ref, rt_ref):
    x = x_ref[...].astype(jnp.bfloat16)
    logits = jax.lax.dot_general(
        x, rw_ref[...].astype(jnp.bfloat16), (((1,), (1,)), ((), ())),
        preferred_element_type=jnp.float32,
    )
    i8 = jax.lax.broadcasted_iota(jnp.int32, (N, E), 1)
    m1 = jnp.max(logits, axis=1, keepdims=True)
    am1 = jnp.min(jnp.where(logits == m1, i8, E), axis=1, keepdims=True)
    lm = jnp.where(i8 == am1, -jnp.inf, logits)
    m2 = jnp.max(lm, axis=1, keepdims=True)
    am2 = jnp.min(jnp.where(lm == m2, i8, E), axis=1, keepdims=True)
    w1 = 1.0 / (1.0 + jnp.exp(m2 - m1))
    w2 = 1.0 - w1
    sel1 = (i8 == am1).astype(jnp.float32) * w1
    sel2 = (i8 == am2).astype(jnp.float32) * w2
    rt_ref[...] = (sel1 + sel2).T.reshape(E, 1, N)


def _router(x, rw):
    return pl.pallas_call(
        _router_body,
        out_shape=jax.ShapeDtypeStruct((E, 1, N), jnp.float32),
    )(x, rw)


def _dense_body(rt_ref, x_ref, g_ref, u_ref, d_ref, out_ref):
    e = pl.program_id(0)
    xb = x_ref[...]
    g = jax.lax.dot_general(xb, g_ref[0], (((1,), (1,)), ((), ())),
                            preferred_element_type=jnp.float32)
    u = jax.lax.dot_general(xb, u_ref[0], (((1,), (1,)), ((), ())),
                            preferred_element_type=jnp.float32)
    h = (g * jax.lax.logistic(g) * u).astype(jnp.bfloat16)
    y = jax.lax.dot_general(h, d_ref[0], (((1,), (1,)), ((), ())),
                            preferred_element_type=jnp.float32)
    we = rt_ref[0, 0, :][:, None]
    acc = y * we

    @pl.when(e == 0)
    def _():
        out_ref[...] = acc

    @pl.when(e > 0)
    def _():
        out_ref[...] += acc


def _dense_experts(rt, x_bf, gate_bf, up_bf, down_bf):
    return pl.pallas_call(
        _dense_body,
        grid=(E,),
        in_specs=[
            pl.BlockSpec((1, 1, N), lambda e: (e, 0, 0)),
            pl.BlockSpec((N, H), lambda e: (0, 0)),
            pl.BlockSpec((1, F, H), lambda e: (e, 0, 0)),
            pl.BlockSpec((1, F, H), lambda e: (e, 0, 0)),
            pl.BlockSpec((1, H, F), lambda e: (e, 0, 0)),
        ],
        out_specs=pl.BlockSpec((N, H), lambda e: (0, 0)),
        out_shape=jax.ShapeDtypeStruct((N, H), jnp.float32),
        compiler_params=pltpu.CompilerParams(
            dimension_semantics=("arbitrary",),
        ),
    )(rt, x_bf, gate_bf, up_bf, down_bf)


def kernel(hidden_states, router_weight, gate_proj, up_proj, down_proj):
    B, T, Hc = hidden_states.shape
    x = hidden_states.reshape(-1, Hc)
    rt = _router(x, router_weight)
    out = _dense_experts(
        rt,
        x.astype(jnp.bfloat16),
        gate_proj.astype(jnp.bfloat16),
        up_proj.astype(jnp.bfloat16),
        down_proj.astype(jnp.bfloat16),
    )
    return out.reshape(B, T, Hc)


# speedup vs baseline: 2.5534x; 1.3717x over previous
import functools

import jax
import jax.numpy as jnp
from jax.experimental import pallas as pl
from jax.experimental.pallas import tpu as pltpu

E = 8
TOP_K = 2
H = 1024
F = 768
N = 2048


def _router_body(x_ref, rw_ref, rt_ref):
    x = x_ref[...].astype(jnp.bfloat16)
    logits = jax.lax.dot_general(
        x, rw_ref[...].astype(jnp.bfloat16), (((1,), (1,)), ((), ())),
        preferred_element_type=jnp.float32,
    )
    i8 = jax.lax.broadcasted_iota(jnp.int32, (N, E), 1)
    m1 = jnp.max(logits, axis=1, keepdims=True)
    am1 = jnp.min(jnp.where(logits == m1, i8, E), axis=1, keepdims=True)
    lm = jnp.where(i8 == am1, -jnp.inf, logits)
    m2 = jnp.max(lm, axis=1, keepdims=True)
    am2 = jnp.min(jnp.where(lm == m2, i8, E), axis=1, keepdims=True)
    w1 = 1.0 / (1.0 + jnp.exp(m2 - m1))
    w2 = 1.0 - w1
    sel1 = (i8 == am1).astype(jnp.float32) * w1
    sel2 = (i8 == am2).astype(jnp.float32) * w2
    rt_ref[...] = (sel1 + sel2).T.reshape(E, 1, N)


def _router(x, rw):
    return pl.pallas_call(
        _router_body,
        out_shape=jax.ShapeDtypeStruct((E, 1, N), jnp.float32),
    )(x, rw)


def _dense_body(rt_ref, x_ref, g_ref, u_ref, d_ref, out_ref):
    e = pl.program_id(0)
    xb = x_ref[...].astype(jnp.bfloat16)
    g = jax.lax.dot_general(xb, g_ref[0].astype(jnp.bfloat16),
                            (((1,), (1,)), ((), ())),
                            preferred_element_type=jnp.float32)
    u = jax.lax.dot_general(xb, u_ref[0].astype(jnp.bfloat16),
                            (((1,), (1,)), ((), ())),
                            preferred_element_type=jnp.float32)
    h = (g * jax.lax.logistic(g) * u).astype(jnp.bfloat16)
    y = jax.lax.dot_general(h, d_ref[0].astype(jnp.bfloat16),
                            (((1,), (1,)), ((), ())),
                            preferred_element_type=jnp.float32)
    we = rt_ref[0, 0, :][:, None]
    acc = y * we

    @pl.when(e == 0)
    def _():
        out_ref[...] = acc

    @pl.when(e > 0)
    def _():
        out_ref[...] += acc


def _dense_experts(rt, x_bf, gate_bf, up_bf, down_bf):
    return pl.pallas_call(
        _dense_body,
        grid=(E,),
        in_specs=[
            pl.BlockSpec((1, 1, N), lambda e: (e, 0, 0)),
            pl.BlockSpec((N, H), lambda e: (0, 0)),
            pl.BlockSpec((1, F, H), lambda e: (e, 0, 0)),
            pl.BlockSpec((1, F, H), lambda e: (e, 0, 0)),
            pl.BlockSpec((1, H, F), lambda e: (e, 0, 0)),
        ],
        out_specs=pl.BlockSpec((N, H), lambda e: (0, 0)),
        out_shape=jax.ShapeDtypeStruct((N, H), jnp.float32),
        compiler_params=pltpu.CompilerParams(
            dimension_semantics=("arbitrary",),
        ),
    )(rt, x_bf, gate_bf, up_bf, down_bf)


def kernel(hidden_states, router_weight, gate_proj, up_proj, down_proj):
    B, T, Hc = hidden_states.shape
    x = hidden_states.reshape(-1, Hc)
    rt = _router(x, router_weight)
    out = _dense_experts(rt, x, gate_proj, up_proj, down_proj)
    return out.reshape(B, T, Hc)
